# baseline (speedup 1.0000x reference)
import functools

import numpy as np
import jax
import jax.numpy as jnp
from jax import lax
from jax.experimental import pallas as pl
from jax.experimental.pallas import tpu as pltpu
from jax.experimental.pallas import tpu_sc as plsc

D_MODEL = 64
SEQ_LEN = 200
NUM_CORES = 2
NUM_SUBCORES = 16
NUM_WORKERS = NUM_CORES * NUM_SUBCORES
LANES = 16
NBUF = 4
LOOKAHEAD = 2


def _positional_encoding(length, dim):
    half = dim // 2
    posn = np.arange(length).reshape(length, 1).astype(np.float32)
    dims = np.arange(half).reshape(1, half).astype(np.float32) / half
    enc = posn / (10000.0 ** dims)
    enc = np.concatenate([np.sin(enc), np.cos(enc)], axis=-1)
    return jnp.asarray(enc, dtype=jnp.float32)


_PE = _positional_encoding(SEQ_LEN, D_MODEL)


@functools.lru_cache(maxsize=None)
def _build(batch):
    seqs_per_worker = batch // NUM_WORKERS
    assert batch % NUM_WORKERS == 0
    n_chunks = seqs_per_worker
    assert n_chunks % NBUF == 0 and n_chunks >= 2 * NBUF
    ids_per_worker = seqs_per_worker * SEQ_LEN
    mesh = plsc.VectorSubcoreMesh(core_axis_name="c", subcore_axis_name="s")

    @functools.partial(
        pl.kernel,
        out_type=jax.ShapeDtypeStruct((batch * SEQ_LEN, D_MODEL), jnp.float32),
        mesh=mesh,
        scratch_types=[
            pltpu.VMEM((ids_per_worker,), jnp.int32),
            pltpu.VMEM((SEQ_LEN, D_MODEL), jnp.float32),
        ]
        + [pltpu.VMEM((SEQ_LEN, D_MODEL), jnp.float32) for _ in range(NBUF)]
        + [pltpu.SemaphoreType.DMA for _ in range(2 * NBUF)],
        compiler_params=pltpu.CompilerParams(use_tc_tiling_on_sc=False),
    )
    def body(ids_hbm, table_hbm, pe_hbm, out_hbm, idx_v, pe_v, *bufs_and_sems):
        rows = list(bufs_and_sems[:NBUF])
        sem_g = list(bufs_and_sems[NBUF : 2 * NBUF])
        sem_w = list(bufs_and_sems[2 * NBUF : 3 * NBUF])

        wid = lax.axis_index("s") * NUM_CORES + lax.axis_index("c")
        w_base = wid * ids_per_worker
        pltpu.sync_copy(pe_hbm, pe_v)
        pltpu.sync_copy(ids_hbm.at[pl.ds(w_base, ids_per_worker)], idx_v)

        def idx_slice(c):
            return idx_v.at[pl.ds(c * SEQ_LEN, SEQ_LEN)]

        def gather_start(c, b):
            pltpu.async_copy(table_hbm.at[idx_slice(c)], rows[b], sem_g[b])

        def gather_wait(c, b):
            pltpu.make_async_copy(
                table_hbm.at[idx_slice(c)], rows[b], sem_g[b]
            ).wait()

        def wb_start(c, b):
            pltpu.async_copy(
                rows[b], out_hbm.at[pl.ds(w_base + c * SEQ_LEN, SEQ_LEN)], sem_w[b]
            )

        def wb_wait(c, b):
            pltpu.make_async_copy(
                rows[b], out_hbm.at[pl.ds(w_base + c * SEQ_LEN, SEQ_LEN)], sem_w[b]
            ).wait()

        def compute(b):
            buf = rows[b]

            def s_body(s, carry):
                for d in range(D_MODEL // LANES):
                    sl = pl.ds(d * LANES, LANES)
                    buf[s, sl] = buf[s, sl] * 9.0 + pe_v[s, sl]
                return carry

            lax.fori_loop(0, SEQ_LEN, s_body, 0, unroll=2)

        for c in range(LOOKAHEAD):
            gather_start(c, c % NBUF)

        for c in range(NBUF):
            b = c % NBUF
            gather_wait(c, b)
            compute(b)
            wb_start(c, b)
            if c - (NBUF - LOOKAHEAD) >= 0:
                wb_wait(c - (NBUF - LOOKAHEAD), (c + LOOKAHEAD) % NBUF)
            gather_start(c + LOOKAHEAD, (c + LOOKAHEAD) % NBUF)

        def super_body(sc, carry):
            for b in range(NBUF):
                c = sc * NBUF + b
                b2 = (b + LOOKAHEAD) % NBUF
                gather_wait(c, b)
                compute(b)
                wb_start(c, b)
                wb_wait(c - (NBUF - LOOKAHEAD), b2)
                gather_start(c + LOOKAHEAD, b2)
            return carry

        lax.fori_loop(1, n_chunks // NBUF - 1, super_body, 0, unroll=False)

        for b in range(NBUF):
            c = n_chunks - NBUF + b
            gather_wait(c, b)
            compute(b)
            wb_start(c, b)
            if c + LOOKAHEAD < n_chunks:
                wb_wait(c - (NBUF - LOOKAHEAD), (c + LOOKAHEAD) % NBUF)
                gather_start(c + LOOKAHEAD, (c + LOOKAHEAD) % NBUF)
        for b in range(NBUF - LOOKAHEAD, NBUF + LOOKAHEAD):
            c = n_chunks - NBUF - LOOKAHEAD + b
            wb_wait(c, c % NBUF)

    return body


@jax.jit
def kernel(input_ids, table):
    batch, seq = input_ids.shape
    ids_flat = input_ids.reshape(batch * seq)
    out = _build(batch)(ids_flat, table, _PE)
    return out.reshape(batch, seq, D_MODEL)
